# baseline (device time: 33977 ns/iter reference)
import functools

import jax
import jax.numpy as jnp
from jax import lax
from jax.experimental import pallas as pl
from jax.experimental.pallas import tpu as pltpu

T = 1024
D = 1024
V_PER = 8192
NQ = 4
Q = T // NQ
C = 4
CR = Q // C
H = C // 2


def _ring_to_xz(t):
    tx = t // 2
    tz = (tx + t) % 2
    return tx, tz


def kernel(ids, E):
    my_x = lax.axis_index("x")
    my_y = lax.axis_index("y")
    my_z = lax.axis_index("z")
    r = 2 * my_x + (my_x + my_z) % 2

    my_ids = lax.dynamic_slice(ids, (r * Q,), (Q,))
    local = my_ids - my_y * V_PER
    mask = (local >= 0) & (local < V_PER)
    safe = jnp.where(mask, local, 0).astype(jnp.int32)
    maskf = mask.astype(jnp.float32)[:, None]

    def body(safe_ref, maskf_ref, e_ref, o_ref, gbuf, rbuf, qbuf,
             gsem, osem, ysend, yrecv, s1rs, s1rr, s1ls, s1lr, s2s, s2r):
        my_x = lax.axis_index("x")
        my_y = lax.axis_index("y")
        my_z = lax.axis_index("z")

        r = 2 * my_x + (my_x + my_z) % 2
        rt = (r + 1) % NQ
        lt = (r + 3) % NQ
        opp = (r + 2) % NQ

        rx, rz = _ring_to_xz(rt)
        lx, lz = _ring_to_xz(lt)
        y_peer = (my_x, 1 - my_y, my_z)
        right = (rx, my_y, rz)
        left = (lx, my_y, lz)

        def issue_gather_chunk(c):
            for i in range(c * CR, (c + 1) * CR):
                pltpu.make_async_copy(
                    e_ref.at[pl.ds(safe_ref[i], 1)],
                    gbuf.at[pl.ds(i, 1)],
                    gsem.at[c],
                ).start()

        issue_gather_chunk(0)
        issue_gather_chunk(1)

        barrier = pltpu.get_barrier_semaphore()
        for dev in (y_peer, right, left):
            pl.semaphore_signal(
                barrier, inc=1, device_id=dev,
                device_id_type=pl.DeviceIdType.MESH,
            )
        pl.semaphore_wait(barrier, 3)

        y_rdmas = []
        for c in range(C):
            if c + 2 < C:
                issue_gather_chunk(c + 2)
            sl = pl.ds(c * CR, CR)
            pltpu.make_async_copy(
                e_ref.at[pl.ds(0, CR)], gbuf.at[sl], gsem.at[c]
            ).wait()
            rd = pltpu.make_async_remote_copy(
                src_ref=gbuf.at[sl],
                dst_ref=rbuf.at[sl],
                send_sem=ysend.at[c],
                recv_sem=yrecv.at[c],
                device_id=y_peer,
                device_id_type=pl.DeviceIdType.MESH,
            )
            rd.start()
            y_rdmas.append(rd)

        s1r_rdmas = []
        s1l_rdmas = []
        for c in range(C):
            sl = pl.ds(c * CR, CR)
            y_rdmas[c].wait_recv()
            qbuf[r, c * CR:(c + 1) * CR, :] = jnp.where(
                maskf_ref[c * CR:(c + 1) * CR, :] != 0.0,
                gbuf[c * CR:(c + 1) * CR, :],
                rbuf[c * CR:(c + 1) * CR, :],
            )
            rdr = pltpu.make_async_remote_copy(
                src_ref=qbuf.at[r, sl],
                dst_ref=qbuf.at[r, sl],
                send_sem=s1rs.at[c],
                recv_sem=s1rr.at[c],
                device_id=right,
                device_id_type=pl.DeviceIdType.MESH,
            )
            rdl = pltpu.make_async_remote_copy(
                src_ref=qbuf.at[r, sl],
                dst_ref=qbuf.at[r, sl],
                send_sem=s1ls.at[c],
                recv_sem=s1lr.at[c],
                device_id=left,
                device_id_type=pl.DeviceIdType.MESH,
            )
            rdr.start()
            rdl.start()
            s1r_rdmas.append(rdr)
            s1l_rdmas.append(rdl)

        copies = []

        def copy_quarter(slot, sem_idx):
            cp = pltpu.make_async_copy(
                qbuf.at[slot],
                o_ref.at[pl.ds(slot * Q, Q)],
                osem.at[sem_idx],
            )
            cp.start()
            copies.append(cp)

        copy_quarter(r, 0)

        s2_rdmas = []
        for c in range(C):
            sl = pl.ds(c * CR, CR)
            if c < H:
                s1r_rdmas[c].wait_recv()
                rd = pltpu.make_async_remote_copy(
                    src_ref=qbuf.at[lt, sl],
                    dst_ref=qbuf.at[lt, sl],
                    send_sem=s2s.at[c],
                    recv_sem=s2r.at[c],
                    device_id=right,
                    device_id_type=pl.DeviceIdType.MESH,
                )
            else:
                s1l_rdmas[c].wait_recv()
                rd = pltpu.make_async_remote_copy(
                    src_ref=qbuf.at[rt, sl],
                    dst_ref=qbuf.at[rt, sl],
                    send_sem=s2s.at[c],
                    recv_sem=s2r.at[c],
                    device_id=left,
                    device_id_type=pl.DeviceIdType.MESH,
                )
            rd.start()
            s2_rdmas.append(rd)

        for c in range(H, C):
            s1r_rdmas[c].wait_recv()
        copy_quarter(lt, 1)
        for c in range(0, H):
            s1l_rdmas[c].wait_recv()
        copy_quarter(rt, 2)
        for c in range(C):
            s2_rdmas[c].wait_recv()
            pltpu.make_async_copy(
                qbuf.at[opp, pl.ds(c * CR, CR)],
                o_ref.at[pl.ds(opp * Q + c * CR, CR)],
                osem.at[3],
            ).start()
        copies.append(
            pltpu.make_async_copy(
                qbuf.at[opp], o_ref.at[pl.ds(opp * Q, Q)], osem.at[3]
            )
        )

        for c in range(C):
            y_rdmas[c].wait_send()
            s1r_rdmas[c].wait_send()
            s1l_rdmas[c].wait_send()
            s2_rdmas[c].wait_send()
        for cp in copies:
            cp.wait()

        @functools.partial(
            pl.run_scoped, exit_sem=pltpu.SemaphoreType.REGULAR
        )
        def _(exit_sem):
            for dev in (y_peer, right, left):
                pl.semaphore_signal(
                    exit_sem, inc=1, device_id=dev,
                    device_id_type=pl.DeviceIdType.MESH,
                )
            pl.semaphore_wait(exit_sem, 3)

    return pl.pallas_call(
        body,
        out_shape=jax.ShapeDtypeStruct((T, D), jnp.float32),
        in_specs=[
            pl.BlockSpec(memory_space=pltpu.SMEM),
            pl.BlockSpec(memory_space=pltpu.VMEM),
            pl.BlockSpec(memory_space=pl.ANY),
        ],
        out_specs=pl.BlockSpec(memory_space=pl.ANY),
        scratch_shapes=[
            pltpu.VMEM((Q, D), jnp.float32),
            pltpu.VMEM((Q, D), jnp.float32),
            pltpu.VMEM((NQ, Q, D), jnp.float32),
            pltpu.SemaphoreType.DMA((C,)),
            pltpu.SemaphoreType.DMA((NQ,)),
            pltpu.SemaphoreType.DMA((C,)),
            pltpu.SemaphoreType.DMA((C,)),
            pltpu.SemaphoreType.DMA((C,)),
            pltpu.SemaphoreType.DMA((C,)),
            pltpu.SemaphoreType.DMA((C,)),
            pltpu.SemaphoreType.DMA((C,)),
            pltpu.SemaphoreType.DMA((C,)),
            pltpu.SemaphoreType.DMA((C,)),
        ],
        compiler_params=pltpu.CompilerParams(collective_id=0),
    )(safe, maskf, E)


# device time: 33827 ns/iter; 1.0044x vs baseline; 1.0044x over previous
import functools

import jax
import jax.numpy as jnp
from jax import lax
from jax.experimental import pallas as pl
from jax.experimental.pallas import tpu as pltpu

T = 1024
D = 1024
V_PER = 8192
NQ = 4
Q = T // NQ
C = 4
CR = Q // C
H = C // 2


def _ring_to_xz(t):
    tx = t // 2
    tz = (tx + t) % 2
    return tx, tz


def kernel(ids, E):
    my_x = lax.axis_index("x")
    my_y = lax.axis_index("y")
    my_z = lax.axis_index("z")
    r = 2 * my_x + (my_x + my_z) % 2

    my_ids = lax.dynamic_slice(ids, (r * Q,), (Q,))
    local = my_ids - my_y * V_PER
    mask = (local >= 0) & (local < V_PER)
    safe = jnp.where(mask, local, 0).astype(jnp.int32)
    maskf = mask.astype(jnp.float32)[:, None]

    def body(safe_ref, maskf_ref, e_ref, o_ref, gbuf, rbuf, qbuf,
             gsem, osem, ysend, yrecv, s1rs, s1rr, s1ls, s1lr, s2s, s2r):
        my_x = lax.axis_index("x")
        my_y = lax.axis_index("y")
        my_z = lax.axis_index("z")

        r = 2 * my_x + (my_x + my_z) % 2
        rt = (r + 1) % NQ
        lt = (r + 3) % NQ
        opp = (r + 2) % NQ

        rx, rz = _ring_to_xz(rt)
        lx, lz = _ring_to_xz(lt)
        y_peer = (my_x, 1 - my_y, my_z)
        right = (rx, my_y, rz)
        left = (lx, my_y, lz)

        def issue_gather_chunk(c):
            for i in range(c * CR, (c + 1) * CR):
                pltpu.make_async_copy(
                    e_ref.at[pl.ds(safe_ref[i], 1)],
                    gbuf.at[pl.ds(i, 1)],
                    gsem.at[c],
                ).start()

        issue_gather_chunk(0)

        barrier = pltpu.get_barrier_semaphore()
        for dev in (y_peer, right, left):
            pl.semaphore_signal(
                barrier, inc=1, device_id=dev,
                device_id_type=pl.DeviceIdType.MESH,
            )
        pl.semaphore_wait(barrier, 3)

        y_rdmas = []
        for c in range(C):
            if c + 1 < C:
                issue_gather_chunk(c + 1)
            sl = pl.ds(c * CR, CR)
            pltpu.make_async_copy(
                e_ref.at[pl.ds(0, CR)], gbuf.at[sl], gsem.at[c]
            ).wait()
            rd = pltpu.make_async_remote_copy(
                src_ref=gbuf.at[sl],
                dst_ref=rbuf.at[sl],
                send_sem=ysend.at[c],
                recv_sem=yrecv.at[c],
                device_id=y_peer,
                device_id_type=pl.DeviceIdType.MESH,
            )
            rd.start()
            y_rdmas.append(rd)

        s1r_rdmas = []
        s1l_rdmas = []
        for c in range(C):
            sl = pl.ds(c * CR, CR)
            y_rdmas[c].wait_recv()
            qbuf[r, c * CR:(c + 1) * CR, :] = jnp.where(
                maskf_ref[c * CR:(c + 1) * CR, :] != 0.0,
                gbuf[c * CR:(c + 1) * CR, :],
                rbuf[c * CR:(c + 1) * CR, :],
            )
            rdr = pltpu.make_async_remote_copy(
                src_ref=qbuf.at[r, sl],
                dst_ref=qbuf.at[r, sl],
                send_sem=s1rs.at[c],
                recv_sem=s1rr.at[c],
                device_id=right,
                device_id_type=pl.DeviceIdType.MESH,
            )
            rdl = pltpu.make_async_remote_copy(
                src_ref=qbuf.at[r, sl],
                dst_ref=qbuf.at[r, sl],
                send_sem=s1ls.at[c],
                recv_sem=s1lr.at[c],
                device_id=left,
                device_id_type=pl.DeviceIdType.MESH,
            )
            rdr.start()
            rdl.start()
            s1r_rdmas.append(rdr)
            s1l_rdmas.append(rdl)

        copies = []

        def copy_quarter(slot, sem_idx):
            cp = pltpu.make_async_copy(
                qbuf.at[slot],
                o_ref.at[pl.ds(slot * Q, Q)],
                osem.at[sem_idx],
            )
            cp.start()
            copies.append(cp)

        copy_quarter(r, 0)

        s2_rdmas = []
        for c in range(C):
            sl = pl.ds(c * CR, CR)
            if c < H:
                s1r_rdmas[c].wait_recv()
                rd = pltpu.make_async_remote_copy(
                    src_ref=qbuf.at[lt, sl],
                    dst_ref=qbuf.at[lt, sl],
                    send_sem=s2s.at[c],
                    recv_sem=s2r.at[c],
                    device_id=right,
                    device_id_type=pl.DeviceIdType.MESH,
                )
            else:
                s1l_rdmas[c].wait_recv()
                rd = pltpu.make_async_remote_copy(
                    src_ref=qbuf.at[rt, sl],
                    dst_ref=qbuf.at[rt, sl],
                    send_sem=s2s.at[c],
                    recv_sem=s2r.at[c],
                    device_id=left,
                    device_id_type=pl.DeviceIdType.MESH,
                )
            rd.start()
            s2_rdmas.append(rd)

        for c in range(H, C):
            s1r_rdmas[c].wait_recv()
        copy_quarter(lt, 1)
        for c in range(0, H):
            s1l_rdmas[c].wait_recv()
        copy_quarter(rt, 2)
        for c in range(C):
            s2_rdmas[c].wait_recv()
        copy_quarter(opp, 3)

        for c in range(C):
            y_rdmas[c].wait_send()
            s1r_rdmas[c].wait_send()
            s1l_rdmas[c].wait_send()
            s2_rdmas[c].wait_send()
        for cp in copies:
            cp.wait()

        @functools.partial(
            pl.run_scoped, exit_sem=pltpu.SemaphoreType.REGULAR
        )
        def _(exit_sem):
            for dev in (y_peer, right, left):
                pl.semaphore_signal(
                    exit_sem, inc=1, device_id=dev,
                    device_id_type=pl.DeviceIdType.MESH,
                )
            pl.semaphore_wait(exit_sem, 3)

    return pl.pallas_call(
        body,
        out_shape=jax.ShapeDtypeStruct((T, D), jnp.float32),
        in_specs=[
            pl.BlockSpec(memory_space=pltpu.SMEM),
            pl.BlockSpec(memory_space=pltpu.VMEM),
            pl.BlockSpec(memory_space=pl.ANY),
        ],
        out_specs=pl.BlockSpec(memory_space=pl.ANY),
        scratch_shapes=[
            pltpu.VMEM((Q, D), jnp.float32),
            pltpu.VMEM((Q, D), jnp.float32),
            pltpu.VMEM((NQ, Q, D), jnp.float32),
            pltpu.SemaphoreType.DMA((C,)),
            pltpu.SemaphoreType.DMA((NQ,)),
            pltpu.SemaphoreType.DMA((C,)),
            pltpu.SemaphoreType.DMA((C,)),
            pltpu.SemaphoreType.DMA((C,)),
            pltpu.SemaphoreType.DMA((C,)),
            pltpu.SemaphoreType.DMA((C,)),
            pltpu.SemaphoreType.DMA((C,)),
            pltpu.SemaphoreType.DMA((C,)),
            pltpu.SemaphoreType.DMA((C,)),
        ],
        compiler_params=pltpu.CompilerParams(collective_id=0),
    )(safe, maskf, E)
